# baseline (speedup 1.0000x reference)
import jax
import jax.numpy as jnp
from jax import lax
from jax.experimental import pallas as pl
from jax.experimental.pallas import tpu as pltpu

TM = 512
NBUF = 4


def _sage_kern(adj_hbm, x_ref, w_ref, o_ref, buf, sem):
    n_rows, S = adj_hbm.shape
    num_chunks = n_rows // TM
    blocks_per_batch = S // TM
    w = w_ref[...]

    def chunk_copy(i, slot):
        return pltpu.make_async_copy(
            adj_hbm.at[pl.ds(i * TM, TM), :],
            buf.at[slot],
            sem.at[slot],
        )

    for k in range(NBUF - 1):
        chunk_copy(k, k).start()

    def compute(i, slot):
        b = lax.div(i, blocks_per_batch)
        xb = x_ref[pl.ds(b * S, S), :]
        a = buf[slot]
        tmp = jnp.dot(a, xb, preferred_element_type=jnp.float32)
        res = tmp + x_ref[pl.ds(i * TM, TM), :]
        o_ref[pl.ds(i * TM, TM), :] = jnp.dot(
            res, w, preferred_element_type=jnp.float32)

    def body(m, _):
        base = m * NBUF
        for u in range(NBUF):
            i = base + u
            chunk_copy(i, u).wait()
            nxt = i + NBUF - 1
            nslot = (nxt) % NBUF
            @pl.when(nxt < num_chunks)
            def _start_next(nxt=nxt, nslot=nslot):
                chunk_copy(nxt, nslot).start()
            compute(i, u)
        return 0

    lax.fori_loop(0, num_chunks // NBUF, body, 0)


def kernel(x, adj_t, W):
    B, S, _ = adj_t.shape
    N, IN = x.shape
    OUT = W.shape[1]
    adj2d = adj_t.reshape(N, S)

    out = pl.pallas_call(
        _sage_kern,
        in_specs=[
            pl.BlockSpec(memory_space=pltpu.MemorySpace.HBM),
            pl.BlockSpec(memory_space=pltpu.MemorySpace.VMEM),
            pl.BlockSpec(memory_space=pltpu.MemorySpace.VMEM),
        ],
        out_specs=pl.BlockSpec(memory_space=pltpu.MemorySpace.VMEM),
        out_shape=jax.ShapeDtypeStruct((N, OUT), jnp.float32),
        scratch_shapes=[
            pltpu.VMEM((NBUF, TM, S), jnp.float32),
            pltpu.SemaphoreType.DMA((NBUF,)),
        ],
    )(adj2d, x, W)
    return out


# speedup vs baseline: 1.0783x; 1.0783x over previous
import jax
import jax.numpy as jnp
from jax import lax
from jax.experimental import pallas as pl
from jax.experimental.pallas import tpu as pltpu

TM = 512
NBUF = 4


def _sage_kern(adj_hbm, x_ref, w_ref, o_ref, buf, sem):
    n_rows, S = adj_hbm.shape
    num_chunks = n_rows // TM
    blocks_per_batch = S // TM
    w = w_ref[...]

    def chunk_copy(i, slot):
        return pltpu.make_async_copy(
            adj_hbm.at[pl.ds(i * TM, TM), :],
            buf.at[slot],
            sem.at[slot],
        )

    for k in range(NBUF - 1):
        chunk_copy(k, k).start()

    def body(i, _):
        slot = lax.rem(i, NBUF)
        chunk_copy(i, slot).wait()
        nxt = i + NBUF - 1
        @pl.when(nxt < num_chunks)
        def _start_next():
            chunk_copy(nxt, lax.rem(nxt, NBUF)).start()
        b = lax.div(i, blocks_per_batch)
        xb = x_ref[pl.ds(b * S, S), :].astype(jnp.bfloat16)
        a = buf[slot].astype(jnp.bfloat16)
        tmp = jnp.dot(a, xb, preferred_element_type=jnp.float32)
        res = tmp + x_ref[pl.ds(i * TM, TM), :]
        o_ref[pl.ds(i * TM, TM), :] = jnp.dot(
            res, w, preferred_element_type=jnp.float32)
        return 0

    lax.fori_loop(0, num_chunks, body, 0)


def kernel(x, adj_t, W):
    B, S, _ = adj_t.shape
    N, IN = x.shape
    OUT = W.shape[1]
    adj2d = adj_t.reshape(N, S)

    out = pl.pallas_call(
        _sage_kern,
        in_specs=[
            pl.BlockSpec(memory_space=pltpu.MemorySpace.HBM),
            pl.BlockSpec(memory_space=pltpu.MemorySpace.VMEM),
            pl.BlockSpec(memory_space=pltpu.MemorySpace.VMEM),
        ],
        out_specs=pl.BlockSpec(memory_space=pltpu.MemorySpace.VMEM),
        out_shape=jax.ShapeDtypeStruct((N, OUT), jnp.float32),
        scratch_shapes=[
            pltpu.VMEM((NBUF, TM, S), jnp.float32),
            pltpu.SemaphoreType.DMA((NBUF,)),
        ],
    )(adj2d, x, W)
    return out


# speedup vs baseline: 1.0790x; 1.0006x over previous
import jax
import jax.numpy as jnp
from jax import lax
from jax.experimental import pallas as pl
from jax.experimental.pallas import tpu as pltpu

TM = 512
NBUF = 4


def _sage_kern(adj_hbm, x_ref, w_ref, o_ref, buf, sem):
    n_rows, S = adj_hbm.shape
    num_chunks = n_rows // TM
    blocks_per_batch = S // TM
    w = w_ref[...]

    def chunk_copy(i, slot):
        return pltpu.make_async_copy(
            adj_hbm.at[pl.ds(i * TM, TM), :],
            buf.at[slot, pl.ds(0, TM)],
            sem.at[slot],
        )

    for k in range(NBUF - 1):
        chunk_copy(k, k).start()

    def body(i, _):
        slot = lax.rem(i, NBUF)
        chunk_copy(i, slot).wait()
        nxt = i + NBUF - 1
        @pl.when(nxt < num_chunks)
        def _start_next():
            chunk_copy(nxt, lax.rem(nxt, NBUF)).start()
        b = lax.div(i, blocks_per_batch)
        xb = x_ref[pl.ds(b * S, S), :]
        a = buf[slot, :TM]
        tmp = jnp.dot(a, xb, preferred_element_type=jnp.float32)
        res = tmp + x_ref[pl.ds(i * TM, TM), :]
        o_ref[pl.ds(i * TM, TM), :] = jnp.dot(
            res, w, preferred_element_type=jnp.float32)
        return 0

    lax.fori_loop(0, num_chunks, body, 0)


def kernel(x, adj_t, W):
    B, S, _ = adj_t.shape
    N, IN = x.shape
    OUT = W.shape[1]
    adj2d = adj_t.reshape(N, S)

    out = pl.pallas_call(
        _sage_kern,
        in_specs=[
            pl.BlockSpec(memory_space=pltpu.MemorySpace.HBM),
            pl.BlockSpec(memory_space=pltpu.MemorySpace.VMEM),
            pl.BlockSpec(memory_space=pltpu.MemorySpace.VMEM),
        ],
        out_specs=pl.BlockSpec(memory_space=pltpu.MemorySpace.VMEM),
        out_shape=jax.ShapeDtypeStruct((N, OUT), jnp.float32),
        scratch_shapes=[
            pltpu.VMEM((NBUF, TM + 8, S), jnp.float32),
            pltpu.SemaphoreType.DMA((NBUF,)),
        ],
    )(adj2d, x, W)
    return out


# speedup vs baseline: 1.4671x; 1.3597x over previous
import jax
import jax.numpy as jnp
from jax import lax
from jax.experimental import pallas as pl
from jax.experimental.pallas import tpu as pltpu

TM = 512
NBUF = 4


def _sage_kern(adj_hbm, xt_ref, w_ref, ot_ref, buf, sem):
    n_rows, S = adj_hbm.shape
    num_chunks = n_rows // TM
    blocks_per_batch = S // TM
    w = w_ref[...]

    def chunk_copy(i, slot):
        return pltpu.make_async_copy(
            adj_hbm.at[pl.ds(i * TM, TM), :],
            buf.at[slot],
            sem.at[slot],
        )

    for k in range(NBUF - 1):
        chunk_copy(k, k).start()

    def body(i, _):
        slot = lax.rem(i, NBUF)
        chunk_copy(i, slot).wait()
        nxt = i + NBUF - 1
        @pl.when(nxt < num_chunks)
        def _start_next():
            chunk_copy(nxt, lax.rem(nxt, NBUF)).start()
        b = lax.div(i, blocks_per_batch)
        xbt = xt_ref[:, pl.ds(b * S, S)]
        a = buf[slot]
        tmpt = lax.dot_general(
            xbt, a, (((1,), (1,)), ((), ())),
            preferred_element_type=jnp.float32)
        rest = tmpt + xt_ref[:, pl.ds(i * TM, TM)]
        ot_ref[:, pl.ds(i * TM, TM)] = lax.dot_general(
            w, rest, (((0,), (0,)), ((), ())),
            preferred_element_type=jnp.float32)
        return 0

    lax.fori_loop(0, num_chunks, body, 0)


def kernel(x, adj_t, W):
    B, S, _ = adj_t.shape
    N, IN = x.shape
    OUT = W.shape[1]
    adj2d = adj_t.reshape(N, S)
    xt = x.T

    outt = pl.pallas_call(
        _sage_kern,
        in_specs=[
            pl.BlockSpec(memory_space=pltpu.MemorySpace.HBM),
            pl.BlockSpec(memory_space=pltpu.MemorySpace.VMEM),
            pl.BlockSpec(memory_space=pltpu.MemorySpace.VMEM),
        ],
        out_specs=pl.BlockSpec(memory_space=pltpu.MemorySpace.VMEM),
        out_shape=jax.ShapeDtypeStruct((OUT, N), jnp.float32),
        scratch_shapes=[
            pltpu.VMEM((NBUF, TM, S), jnp.float32),
            pltpu.SemaphoreType.DMA((NBUF,)),
        ],
    )(adj2d, xt, W)
    return outt.T
